# baseline (device time: 25885 ns/iter reference)
import jax
import jax.numpy as jnp
from jax import lax
from jax.experimental import pallas as pl
from jax.experimental.pallas import tpu as pltpu

N_DEV = 8
M_PER = 128
K = 1024
N_PER = 128

N_SUB = 4
SUB_ROWS = M_PER // N_SUB

_GELU_C = 0.7978845608028654


def _gelu(y):
    return 0.5 * y * (1.0 + jnp.tanh(_GELU_C * (y + 0.044715 * y * y * y)))


def _ring(s):
    s = s % N_DEV
    return jnp.where(s < 4, s, 11 - s)


def kernel(x, w_mat):
    def body(x_ref, w_ref, out_ref, comm_ref,
             h_send, h_recv, l_send, l_recv, c_send, c_recv):
        my_pos = lax.axis_index("i")
        my_slot = _ring(my_pos)
        sign = 1 - 2 * (my_slot % 2)

        partner = {
            "h": _ring(my_slot - sign),
            "l": _ring(my_slot + sign),
            "c": _ring(my_slot + 3 * sign),
        }
        sems = {"h": (h_send, h_recv), "l": (l_send, l_recv),
                "c": (c_send, c_recv)}
        o_h1, o_h2, o_h3 = (partner["h"], _ring(my_slot - 2 * sign),
                            _ring(my_slot - 3 * sign))
        o_l1, o_l2 = partner["l"], _ring(my_slot + 2 * sign)
        o_c1, o_c2 = partner["c"], _ring(my_slot + 4 * sign)

        barrier_sem = pltpu.get_barrier_semaphore()
        for nbr in partner.values():
            pl.semaphore_signal(
                barrier_sem, inc=1,
                device_id=(nbr,), device_id_type=pl.DeviceIdType.MESH,
            )
        pl.semaphore_wait(barrier_sem, 3)

        pending_sends = []

        def start_send(link, depth, origin, sub, src=None):
            send_sem, recv_sem = sems[link]
            desc = pltpu.make_async_remote_copy(
                src_ref=comm_ref.at[origin, sub] if src is None else src,
                dst_ref=comm_ref.at[origin, sub],
                send_sem=send_sem.at[depth, sub],
                recv_sem=recv_sem.at[depth, sub],
                device_id=(partner[link],),
                device_id_type=pl.DeviceIdType.MESH,
            )
            desc.start()
            pending_sends.append(desc)

        def wait_recv(link, depth, origin, sub):
            send_sem, recv_sem = sems[link]
            desc = pltpu.make_async_remote_copy(
                src_ref=comm_ref.at[origin, sub],
                dst_ref=comm_ref.at[origin, sub],
                send_sem=send_sem.at[depth, sub],
                recv_sem=recv_sem.at[depth, sub],
                device_id=(partner[link],),
                device_id_type=pl.DeviceIdType.MESH,
            )
            desc.wait_recv()

        def chord_send_full(depth, origin, src=None):
            desc = pltpu.make_async_remote_copy(
                src_ref=comm_ref.at[origin] if src is None else src,
                dst_ref=comm_ref.at[origin],
                send_sem=c_send.at[depth, 0],
                recv_sem=c_recv.at[depth, 0],
                device_id=(partner["c"],),
                device_id_type=pl.DeviceIdType.MESH,
            )
            desc.start()
            pending_sends.append(desc)

        def chord_wait_full(depth, origin):
            desc = pltpu.make_async_remote_copy(
                src_ref=comm_ref.at[origin],
                dst_ref=comm_ref.at[origin],
                send_sem=c_send.at[depth, 0],
                recv_sem=c_recv.at[depth, 0],
                device_id=(partner["c"],),
                device_id_type=pl.DeviceIdType.MESH,
            )
            desc.wait_recv()

        def gemm(origin):
            y = jnp.dot(comm_ref[origin].reshape(M_PER, K), w_ref[...],
                        preferred_element_type=jnp.float32)
            out_ref[pl.ds(origin * M_PER, M_PER), :] = _gelu(y)

        for j in range(N_SUB):
            xsub = x_ref.at[pl.ds(j * SUB_ROWS, SUB_ROWS)]
            start_send("h", 0, my_pos, j, src=xsub)
            start_send("l", 0, my_pos, j, src=xsub)
            start_send("c", 0, my_pos, j, src=xsub)

        y = jnp.dot(x_ref[...], w_ref[...], preferred_element_type=jnp.float32)
        out_ref[pl.ds(my_pos * M_PER, M_PER), :] = _gelu(y)

        for j in range(N_SUB):
            wait_recv("l", 0, o_l1, j)
            start_send("h", 1, o_l1, j)
            wait_recv("h", 0, o_h1, j)
            start_send("l", 1, o_h1, j)
        chord_send_full(1, o_h1)

        for j in range(N_SUB):
            wait_recv("l", 1, o_l2, j)
            start_send("h", 2, o_l2, j)

        gemm(o_l1)
        gemm(o_h1)
        for j in range(N_SUB):
            wait_recv("c", 0, o_c1, j)
        gemm(o_c1)
        gemm(o_l2)
        for j in range(N_SUB):
            wait_recv("h", 1, o_h2, j)
        gemm(o_h2)
        chord_wait_full(1, o_c2)
        gemm(o_c2)
        for j in range(N_SUB):
            wait_recv("h", 2, o_h3, j)
        gemm(o_h3)

        for desc in pending_sends:
            desc.wait_send()

    return pl.pallas_call(
        body,
        out_shape=jax.ShapeDtypeStruct((N_DEV * M_PER, N_PER), jnp.float32),
        in_specs=[
            pl.BlockSpec(memory_space=pltpu.VMEM),
            pl.BlockSpec(memory_space=pltpu.VMEM),
        ],
        out_specs=pl.BlockSpec(memory_space=pltpu.VMEM),
        scratch_shapes=[
            pltpu.VMEM((N_DEV, N_SUB, SUB_ROWS, K), jnp.float32),
            pltpu.SemaphoreType.DMA((3, N_SUB)),
            pltpu.SemaphoreType.DMA((3, N_SUB)),
            pltpu.SemaphoreType.DMA((2, N_SUB)),
            pltpu.SemaphoreType.DMA((2, N_SUB)),
            pltpu.SemaphoreType.DMA((2, N_SUB)),
            pltpu.SemaphoreType.DMA((2, N_SUB)),
        ],
        compiler_params=pltpu.CompilerParams(collective_id=0),
    )(x, w_mat)


# device time: 25700 ns/iter; 1.0072x vs baseline; 1.0072x over previous
import jax
import jax.numpy as jnp
from jax import lax
from jax.experimental import pallas as pl
from jax.experimental.pallas import tpu as pltpu

N_DEV = 8
M_PER = 128
K = 1024
N_PER = 128

N_SUB = 4
SUB_ROWS = M_PER // N_SUB

_GELU_C = 0.7978845608028654


def _gelu(y):
    return 0.5 * y * (1.0 + jnp.tanh(_GELU_C * (y + 0.044715 * y * y * y)))


def _ring(s):
    s = s % N_DEV
    return jnp.where(s < 4, s, 11 - s)


def kernel(x, w_mat):
    def body(x_ref, w_ref, out_ref, comm_ref,
             h_send, h_recv, l_send, l_recv, c_send, c_recv):
        my_pos = lax.axis_index("i")
        my_slot = _ring(my_pos)
        sign = 1 - 2 * (my_slot % 2)

        partner = {
            "h": _ring(my_slot - sign),
            "l": _ring(my_slot + sign),
            "c": _ring(my_slot + 3 * sign),
        }
        sems = {"h": (h_send, h_recv), "l": (l_send, l_recv),
                "c": (c_send, c_recv)}
        o_h1, o_h2, o_h3 = (partner["h"], _ring(my_slot - 2 * sign),
                            _ring(my_slot - 3 * sign))
        o_l1, o_l2 = partner["l"], _ring(my_slot + 2 * sign)
        o_c1, o_c2 = partner["c"], _ring(my_slot + 4 * sign)

        barrier_sem = pltpu.get_barrier_semaphore()
        for nbr in partner.values():
            pl.semaphore_signal(
                barrier_sem, inc=1,
                device_id=(nbr,), device_id_type=pl.DeviceIdType.MESH,
            )
        pl.semaphore_wait(barrier_sem, 3)

        pending_sends = []

        def start_send(link, depth, origin, sub, src=None):
            send_sem, recv_sem = sems[link]
            desc = pltpu.make_async_remote_copy(
                src_ref=comm_ref.at[origin, sub] if src is None else src,
                dst_ref=comm_ref.at[origin, sub],
                send_sem=send_sem.at[depth, sub],
                recv_sem=recv_sem.at[depth, sub],
                device_id=(partner[link],),
                device_id_type=pl.DeviceIdType.MESH,
            )
            desc.start()
            pending_sends.append(desc)

        def wait_recv(link, depth, origin, sub):
            send_sem, recv_sem = sems[link]
            desc = pltpu.make_async_remote_copy(
                src_ref=comm_ref.at[origin, sub],
                dst_ref=comm_ref.at[origin, sub],
                send_sem=send_sem.at[depth, sub],
                recv_sem=recv_sem.at[depth, sub],
                device_id=(partner[link],),
                device_id_type=pl.DeviceIdType.MESH,
            )
            desc.wait_recv()

        def chord_send_full(depth, origin, src=None):
            desc = pltpu.make_async_remote_copy(
                src_ref=comm_ref.at[origin] if src is None else src,
                dst_ref=comm_ref.at[origin],
                send_sem=c_send.at[depth, 0],
                recv_sem=c_recv.at[depth, 0],
                device_id=(partner["c"],),
                device_id_type=pl.DeviceIdType.MESH,
            )
            desc.start()
            pending_sends.append(desc)

        def chord_wait_full(depth, origin):
            desc = pltpu.make_async_remote_copy(
                src_ref=comm_ref.at[origin],
                dst_ref=comm_ref.at[origin],
                send_sem=c_send.at[depth, 0],
                recv_sem=c_recv.at[depth, 0],
                device_id=(partner["c"],),
                device_id_type=pl.DeviceIdType.MESH,
            )
            desc.wait_recv()

        def gemm(origin):
            y = jnp.dot(comm_ref[origin].reshape(M_PER, K), w_ref[...],
                        preferred_element_type=jnp.float32)
            out_ref[pl.ds(origin * M_PER, M_PER), :] = _gelu(y)

        for j in range(N_SUB):
            xsub = x_ref.at[pl.ds(j * SUB_ROWS, SUB_ROWS)]
            start_send("h", 0, my_pos, j, src=xsub)
            start_send("l", 0, my_pos, j, src=xsub)
            start_send("c", 0, my_pos, j, src=xsub)

        y = jnp.dot(x_ref[...], w_ref[...], preferred_element_type=jnp.float32)
        out_ref[pl.ds(my_pos * M_PER, M_PER), :] = _gelu(y)

        for j in range(N_SUB):
            wait_recv("l", 0, o_l1, j)
            start_send("h", 1, o_l1, j)
            wait_recv("h", 0, o_h1, j)
            start_send("l", 1, o_h1, j)
        chord_send_full(1, o_h1)

        for j in range(N_SUB):
            wait_recv("l", 1, o_l2, j)
            start_send("h", 2, o_l2, j)

        gemm(o_l1)
        gemm(o_h1)
        for j in range(N_SUB):
            wait_recv("c", 0, o_c1, j)
        gemm(o_c1)
        gemm(o_l2)
        for j in range(N_SUB):
            wait_recv("h", 1, o_h2, j)
        gemm(o_h2)
        chord_wait_full(1, o_c2)
        gemm(o_c2)
        for j in range(N_SUB):
            wait_recv("h", 2, o_h3, j)
            y = jnp.dot(comm_ref[o_h3, j], w_ref[...],
                        preferred_element_type=jnp.float32)
            out_ref[pl.ds(o_h3 * M_PER + j * SUB_ROWS, SUB_ROWS), :] = _gelu(y)

        for desc in pending_sends:
            desc.wait_send()

    return pl.pallas_call(
        body,
        out_shape=jax.ShapeDtypeStruct((N_DEV * M_PER, N_PER), jnp.float32),
        in_specs=[
            pl.BlockSpec(memory_space=pltpu.VMEM),
            pl.BlockSpec(memory_space=pltpu.VMEM),
        ],
        out_specs=pl.BlockSpec(memory_space=pltpu.VMEM),
        scratch_shapes=[
            pltpu.VMEM((N_DEV, N_SUB, SUB_ROWS, K), jnp.float32),
            pltpu.SemaphoreType.DMA((3, N_SUB)),
            pltpu.SemaphoreType.DMA((3, N_SUB)),
            pltpu.SemaphoreType.DMA((2, N_SUB)),
            pltpu.SemaphoreType.DMA((2, N_SUB)),
            pltpu.SemaphoreType.DMA((2, N_SUB)),
            pltpu.SemaphoreType.DMA((2, N_SUB)),
        ],
        compiler_params=pltpu.CompilerParams(collective_id=0),
    )(x, w_mat)
